# Initial kernel scaffold; baseline (speedup 1.0000x reference)
import jax
import jax.numpy as jnp
from jax.experimental import pallas as pl


def kernel(pillar_features, coords):
    raise NotImplementedError("write your pallas kernel here")


# speedup vs baseline: 6.1001x; 6.1001x over previous
import functools

import jax
import jax.numpy as jnp
from jax import lax
from jax.experimental import pallas as pl
from jax.experimental.pallas import tpu as pltpu
from jax.experimental.pallas import tpu_sc as plsc

NX = 625
NY = 625
NCELL = NX * NY
B = 4
F = 64
P = 12000
PP = 12032

_info = plsc.get_sparse_core_info()
NC = _info.num_cores
NS = _info.num_subcores
L = _info.num_lanes
NW = NC * NS

FG = 8
NFG = F // FG
CHUNK = 12288
SUB = 512
NSUB = CHUNK // SUB
SENT = P
CCH = PP // 2
NCELL_PAD = NW * CHUNK

_mesh = plsc.VectorSubcoreMesh(core_axis_name="c", subcore_axis_name="s")


@functools.partial(
    pl.kernel,
    mesh=_mesh,
    out_type=jax.ShapeDtypeStruct((B, NFG, FG, NCELL_PAD), jnp.float32),
    compiler_params=pltpu.CompilerParams(needs_layout_passes=False),
    scratch_types=[
        pltpu.VMEM((CHUNK,), jnp.int32),
        pltpu.VMEM((CCH,), jnp.int32),
        pltpu.VMEM((CCH,), jnp.int32),
        pltpu.VMEM((FG, PP), jnp.float32),
        pltpu.VMEM((FG, SUB), jnp.float32),
    ],
)
def _sc_scatter(feat_hbm, xs_hbm, ys_hbm, out_hbm, w_ref, cx_ref, cy_ref,
                rows_ref, o_ref):
    wid = lax.axis_index("s") * NC + lax.axis_index("c")
    lo = pl.multiple_of(wid * CHUNK, 128)

    lanes = lax.iota(jnp.int32, L)
    sent_v = jnp.full((L,), SENT, jnp.int32)
    fvecs = [jnp.full((L,), f_in, jnp.int32) for f_in in range(FG)]

    for b in range(B):
        def reset(i, carry):
            w_ref[pl.ds(i * L, L)] = sent_v
            return carry
        lax.fori_loop(0, CHUNK // L, reset, 0)

        def scan_chunk(c, carry):
            base = pl.multiple_of(c * CCH, 128)
            pltpu.sync_copy(xs_hbm.at[b, 0, pl.ds(base, CCH)], cx_ref)
            pltpu.sync_copy(ys_hbm.at[b, 0, pl.ds(base, CCH)], cy_ref)

            def scan_pillars(i, c2):
                xv = cx_ref[pl.ds(i * L, L)]
                yv = cy_ref[pl.ds(i * L, L)]
                local = yv * NX + xv - lo
                pv = lanes + (base + i * L)
                mask = (local >= 0) & (local < CHUNK) & (pv < P)
                local_c = jnp.where(mask, local, 0)
                plsc.store_scatter(w_ref, [local_c], pv, mask=mask)
                return c2
            lax.fori_loop(0, CCH // L, scan_pillars, 0)
            return carry
        lax.fori_loop(0, PP // CCH, scan_chunk, 0)

        def per_group(fg, carry):
            pltpu.sync_copy(feat_hbm.at[b, fg], rows_ref)

            def per_sub(sc, c2):
                off = pl.multiple_of(lo + sc * SUB, 128)

                def gather(j, c3):
                    idx = w_ref[pl.ds(sc * SUB + j * L, L)]
                    for f_in in range(FG):
                        o_ref[f_in, pl.ds(j * L, L)] = plsc.load_gather(
                            rows_ref, [fvecs[f_in], idx])
                    return c3
                lax.fori_loop(0, SUB // L, gather, 0)

                pltpu.sync_copy(o_ref, out_hbm.at[b, fg, :, pl.ds(off, SUB)])
                return c2
            lax.fori_loop(0, NSUB, per_sub, 0)
            return carry
        lax.fori_loop(0, NFG, per_group, 0)


def kernel(pillar_features, coords):
    feat4 = jnp.pad(pillar_features, ((0, 0), (0, 0), (0, PP - P)))
    feat4 = feat4.reshape(B, NFG, FG, PP)
    cpad = jnp.pad(coords, ((0, 0), (0, PP - P), (0, 0)))
    xs = cpad[:, :, 0].astype(jnp.int32).reshape(B, 1, PP)
    ys = cpad[:, :, 1].astype(jnp.int32).reshape(B, 1, PP)
    out = _sc_scatter(feat4, xs, ys)
    return out.reshape(B, F, NCELL_PAD)[:, :, :NCELL].reshape(B, F, NY, NX)
